# baseline (device time: 317297 ns/iter reference)
import jax
import jax.numpy as jnp
from jax import lax
from jax.experimental import pallas as pl
from jax.experimental.pallas import tpu as pltpu

N_DEV = 4
M, K_SH, N = 4096, 1024, 2048
CH = M // N_DEV
NH = N // 2
NSUB = 4
SUB = CH // NSUB
NHOP = N_DEV - 1


def kernel(x, w_mat, scale_x, scale_w):
    def body(x_ref, w_ref, sx_ref, sw_ref, out_ref,
             rsA_buf, rsB_buf,
             rsA_s, rsA_r, rsB_s, rsB_r,
             agA_s, agA_r, agB_s, agB_r):
        my = lax.axis_index("i")
        right = lax.rem(my + 1, N_DEV)
        left = lax.rem(my + N_DEV - 1, N_DEV)

        barrier_sem = pltpu.get_barrier_semaphore()
        for nbr in (left, right):
            pl.semaphore_signal(barrier_sem, inc=1, device_id=(nbr,),
                                device_id_type=pl.DeviceIdType.MESH)
        pl.semaphore_wait(barrier_sem, 2)

        def rows(c, j):
            return pl.ds(c * CH + j * SUB, SUB)

        def rsA_d(h, j):
            sc = lax.rem(my - h + 2 * N_DEV, N_DEV)
            return pltpu.make_async_remote_copy(
                src_ref=out_ref.at[rows(sc, j), 0:NH],
                dst_ref=rsA_buf.at[h, j],
                send_sem=rsA_s.at[h, j], recv_sem=rsA_r.at[h, j],
                device_id=(right,), device_id_type=pl.DeviceIdType.MESH)

        def rsB_d(h, j):
            sc = lax.rem(my + h, N_DEV)
            return pltpu.make_async_remote_copy(
                src_ref=out_ref.at[rows(sc, j), NH:N],
                dst_ref=rsB_buf.at[h, j],
                send_sem=rsB_s.at[h, j], recv_sem=rsB_r.at[h, j],
                device_id=(left,), device_id_type=pl.DeviceIdType.MESH)

        def agA_d(h, j):
            gc = lax.rem(my + 1 - h + 2 * N_DEV, N_DEV)
            return pltpu.make_async_remote_copy(
                src_ref=out_ref.at[rows(gc, j), 0:NH],
                dst_ref=out_ref.at[rows(gc, j), 0:NH],
                send_sem=agA_s.at[h, j], recv_sem=agA_r.at[h, j],
                device_id=(right,), device_id_type=pl.DeviceIdType.MESH)

        def agB_d(h, j):
            gc = lax.rem(my + N_DEV - 1 + h, N_DEV)
            return pltpu.make_async_remote_copy(
                src_ref=out_ref.at[rows(gc, j), NH:N],
                dst_ref=out_ref.at[rows(gc, j), NH:N],
                send_sem=agB_s.at[h, j], recv_sem=agB_r.at[h, j],
                device_id=(left,), device_id_type=pl.DeviceIdType.MESH)

        rsA = {(h, j): rsA_d(h, j) for h in range(NHOP) for j in range(NSUB)}
        rsB = {(h, j): rsB_d(h, j) for h in range(NHOP) for j in range(NSUB)}
        agA = {(h, j): agA_d(h, j) for h in range(NHOP) for j in range(NSUB)}
        agB = {(h, j): agB_d(h, j) for h in range(NHOP) for j in range(NSUB)}

        s = sx_ref[0] * sw_ref[0]

        def compute_chunk(c):
            acc = lax.dot_general(
                x_ref[pl.ds(c * CH, CH), :], w_ref[:, :],
                (((1,), (0,)), ((), ())),
                preferred_element_type=jnp.int32)
            out_ref[pl.ds(c * CH, CH), :] = acc.astype(jnp.float32)

        compute_chunk(my)
        for j in range(NSUB):
            rsA[0, j].start()
            rsB[0, j].start()
        for d in (N_DEV - 1, 1, 2):
            compute_chunk(lax.rem(my + d, N_DEV))

        for h in range(NHOP):
            rcA = lax.rem(my - h - 1 + 2 * N_DEV, N_DEV)
            rcB = lax.rem(my + h + 1, N_DEV)
            for j in range(NSUB):
                rsA[h, j].wait_recv()
                accA = out_ref[rows(rcA, j), 0:NH] + rsA_buf[h, j]
                if h < NHOP - 1:
                    out_ref[rows(rcA, j), 0:NH] = accA
                    rsA[h + 1, j].start()
                else:
                    out_ref[rows(rcA, j), 0:NH] = accA * s
                    agA[0, j].start()
                rsB[h, j].wait_recv()
                accB = out_ref[rows(rcB, j), NH:N] + rsB_buf[h, j]
                if h < NHOP - 1:
                    out_ref[rows(rcB, j), NH:N] = accB
                    rsB[h + 1, j].start()
                else:
                    out_ref[rows(rcB, j), NH:N] = accB * s
                    agB[0, j].start()

        for h in range(NHOP):
            for j in range(NSUB):
                agA[h, j].wait_recv()
                if h < NHOP - 1:
                    agA[h + 1, j].start()
                agB[h, j].wait_recv()
                if h < NHOP - 1:
                    agB[h + 1, j].start()

        for dmap in (rsA, rsB, agA, agB):
            for d in dmap.values():
                d.wait_send()

    return pl.pallas_call(
        body,
        out_shape=jax.ShapeDtypeStruct((M, N), jnp.float32),
        in_specs=[
            pl.BlockSpec(memory_space=pltpu.VMEM),
            pl.BlockSpec(memory_space=pltpu.VMEM),
            pl.BlockSpec(memory_space=pltpu.SMEM),
            pl.BlockSpec(memory_space=pltpu.SMEM),
        ],
        out_specs=pl.BlockSpec(memory_space=pltpu.VMEM),
        scratch_shapes=[
            pltpu.VMEM((NHOP, NSUB, SUB, NH), jnp.float32),
            pltpu.VMEM((NHOP, NSUB, SUB, NH), jnp.float32),
            pltpu.SemaphoreType.DMA((NHOP, NSUB)),
            pltpu.SemaphoreType.DMA((NHOP, NSUB)),
            pltpu.SemaphoreType.DMA((NHOP, NSUB)),
            pltpu.SemaphoreType.DMA((NHOP, NSUB)),
            pltpu.SemaphoreType.DMA((NHOP, NSUB)),
            pltpu.SemaphoreType.DMA((NHOP, NSUB)),
            pltpu.SemaphoreType.DMA((NHOP, NSUB)),
            pltpu.SemaphoreType.DMA((NHOP, NSUB)),
        ],
        compiler_params=pltpu.CompilerParams(
            collective_id=0, vmem_limit_bytes=100 * 1024 * 1024),
    )(x, w_mat, scale_x, scale_w)


# device time: 307740 ns/iter; 1.0311x vs baseline; 1.0311x over previous
import jax
import jax.numpy as jnp
from jax import lax
from jax.experimental import pallas as pl
from jax.experimental.pallas import tpu as pltpu

N_DEV = 4
M, K_SH, N = 4096, 1024, 2048
CH = M // N_DEV
NH = N // 2
NSUB = 2
SUB = CH // NSUB
NHOP = N_DEV - 1


def kernel(x, w_mat, scale_x, scale_w):
    def body(x_ref, w_ref, sx_ref, sw_ref, out_hbm,
             acc, rsA_buf, rsB_buf,
             rsA_s, rsA_r, rsB_s, rsB_r,
             agA_s, agA_r, agB_s, agB_r, hbm_sems):
        my = lax.axis_index("i")
        right = lax.rem(my + 1, N_DEV)
        left = lax.rem(my + N_DEV - 1, N_DEV)

        barrier_sem = pltpu.get_barrier_semaphore()
        for nbr in (left, right):
            pl.semaphore_signal(barrier_sem, inc=1, device_id=(nbr,),
                                device_id_type=pl.DeviceIdType.MESH)
        pl.semaphore_wait(barrier_sem, 2)

        def rows(c, j):
            return pl.ds(c * CH + j * SUB, SUB)

        def rsA_d(h, j):
            sc = lax.rem(my - h + 2 * N_DEV, N_DEV)
            return pltpu.make_async_remote_copy(
                src_ref=acc.at[rows(sc, j), 0:NH],
                dst_ref=rsA_buf.at[h, j],
                send_sem=rsA_s.at[h, j], recv_sem=rsA_r.at[h, j],
                device_id=(right,), device_id_type=pl.DeviceIdType.MESH)

        def rsB_d(h, j):
            sc = lax.rem(my + h, N_DEV)
            return pltpu.make_async_remote_copy(
                src_ref=acc.at[rows(sc, j), NH:N],
                dst_ref=rsB_buf.at[h, j],
                send_sem=rsB_s.at[h, j], recv_sem=rsB_r.at[h, j],
                device_id=(left,), device_id_type=pl.DeviceIdType.MESH)

        def agA_d(h, j):
            gc = lax.rem(my + 1 - h + 2 * N_DEV, N_DEV)
            return pltpu.make_async_remote_copy(
                src_ref=acc.at[rows(gc, j), 0:NH],
                dst_ref=acc.at[rows(gc, j), 0:NH],
                send_sem=agA_s.at[h, j], recv_sem=agA_r.at[h, j],
                device_id=(right,), device_id_type=pl.DeviceIdType.MESH)

        def agB_d(h, j):
            gc = lax.rem(my + N_DEV - 1 + h, N_DEV)
            return pltpu.make_async_remote_copy(
                src_ref=acc.at[rows(gc, j), NH:N],
                dst_ref=acc.at[rows(gc, j), NH:N],
                send_sem=agB_s.at[h, j], recv_sem=agB_r.at[h, j],
                device_id=(left,), device_id_type=pl.DeviceIdType.MESH)

        rsA = {(h, j): rsA_d(h, j) for h in range(NHOP) for j in range(NSUB)}
        rsB = {(h, j): rsB_d(h, j) for h in range(NHOP) for j in range(NSUB)}
        agA = {(h, j): agA_d(h, j) for h in range(NHOP) for j in range(NSUB)}
        agB = {(h, j): agB_d(h, j) for h in range(NHOP) for j in range(NSUB)}

        hbm_copies = []

        def store_final(d, slot, c, j):
            col = slice(0, NH) if d == 0 else slice(NH, N)
            cp = pltpu.make_async_copy(
                acc.at[rows(c, j), col],
                out_hbm.at[rows(c, j), col],
                hbm_sems.at[d, slot, j])
            cp.start()
            hbm_copies.append(cp)

        s = sx_ref[0] * sw_ref[0]

        def compute_chunk(c):
            g = lax.dot_general(
                x_ref[pl.ds(c * CH, CH), :], w_ref[:, :],
                (((1,), (0,)), ((), ())),
                preferred_element_type=jnp.int32)
            acc[pl.ds(c * CH, CH), :] = g.astype(jnp.float32)

        compute_chunk(my)
        for j in range(NSUB):
            rsA[0, j].start()
            rsB[0, j].start()
        for d in (N_DEV - 1, 1, 2):
            compute_chunk(lax.rem(my + d, N_DEV))

        for h in range(NHOP):
            rcA = lax.rem(my - h - 1 + 2 * N_DEV, N_DEV)
            rcB = lax.rem(my + h + 1, N_DEV)
            for j in range(NSUB):
                rsA[h, j].wait_recv()
                accA = acc[rows(rcA, j), 0:NH] + rsA_buf[h, j]
                if h < NHOP - 1:
                    acc[rows(rcA, j), 0:NH] = accA
                    rsA[h + 1, j].start()
                else:
                    acc[rows(rcA, j), 0:NH] = accA * s
                    agA[0, j].start()
                    store_final(0, 0, rcA, j)
                rsB[h, j].wait_recv()
                accB = acc[rows(rcB, j), NH:N] + rsB_buf[h, j]
                if h < NHOP - 1:
                    acc[rows(rcB, j), NH:N] = accB
                    rsB[h + 1, j].start()
                else:
                    acc[rows(rcB, j), NH:N] = accB * s
                    agB[0, j].start()
                    store_final(1, 0, rcB, j)

        for h in range(NHOP):
            rvA = lax.rem(my - h + 2 * N_DEV, N_DEV)
            rvB = lax.rem(my + h, N_DEV)
            for j in range(NSUB):
                agA[h, j].wait_recv()
                if h < NHOP - 1:
                    agA[h + 1, j].start()
                store_final(0, h + 1, rvA, j)
                agB[h, j].wait_recv()
                if h < NHOP - 1:
                    agB[h + 1, j].start()
                store_final(1, h + 1, rvB, j)

        for cp in hbm_copies:
            cp.wait()
        for dmap in (rsA, rsB, agA, agB):
            for dsc in dmap.values():
                dsc.wait_send()

    return pl.pallas_call(
        body,
        out_shape=jax.ShapeDtypeStruct((M, N), jnp.float32),
        in_specs=[
            pl.BlockSpec(memory_space=pltpu.VMEM),
            pl.BlockSpec(memory_space=pltpu.VMEM),
            pl.BlockSpec(memory_space=pltpu.SMEM),
            pl.BlockSpec(memory_space=pltpu.SMEM),
        ],
        out_specs=pl.BlockSpec(memory_space=pl.ANY),
        scratch_shapes=[
            pltpu.VMEM((M, N), jnp.float32),
            pltpu.VMEM((NHOP, NSUB, SUB, NH), jnp.float32),
            pltpu.VMEM((NHOP, NSUB, SUB, NH), jnp.float32),
            pltpu.SemaphoreType.DMA((NHOP, NSUB)),
            pltpu.SemaphoreType.DMA((NHOP, NSUB)),
            pltpu.SemaphoreType.DMA((NHOP, NSUB)),
            pltpu.SemaphoreType.DMA((NHOP, NSUB)),
            pltpu.SemaphoreType.DMA((NHOP, NSUB)),
            pltpu.SemaphoreType.DMA((NHOP, NSUB)),
            pltpu.SemaphoreType.DMA((NHOP, NSUB)),
            pltpu.SemaphoreType.DMA((NHOP, NSUB)),
            pltpu.SemaphoreType.DMA((2, N_DEV, NSUB)),
        ],
        compiler_params=pltpu.CompilerParams(
            collective_id=0, vmem_limit_bytes=100 * 1024 * 1024),
    )(x, w_mat, scale_x, scale_w)
